# Initial kernel scaffold; baseline (speedup 1.0000x reference)
import jax
import jax.numpy as jnp
from jax.experimental import pallas as pl


def kernel(src, idx_qw, idx_qb, idx_kw, idx_kb, idx_ww, idx_wb, in_w, in_b, out_w, out_b, lin1_w, lin1_b, lin2_w, lin2_b, ln1_g, ln1_b, ln2_g, ln2_b):
    raise NotImplementedError("write your pallas kernel here")


# speedup vs baseline: 4.9440x; 4.9440x over previous
import math

import jax
import jax.numpy as jnp
from jax.experimental import pallas as pl
from jax.experimental.pallas import tpu as pltpu

NH = 16
TOPK_K = 64
DX = 32
IH = 4
EPS = 1e-5
NEG_INF = float("-inf")


def _dot_t(a, b):
    return jax.lax.dot_general(a, b, (((1,), (1,)), ((), ())),
                               preferred_element_type=jnp.float32)


def _dot(a, b):
    return jax.lax.dot_general(a, b, (((1,), (0,)), ((), ())),
                               preferred_element_type=jnp.float32)


def _idx_proj_body(x_ref, qw_ref, qb_ref, kw_ref, kb_ref, ww_ref, wb_ref,
                   qo_ref, ko_ref, wo_ref):
    x = x_ref[...]
    qo_ref[...] = _dot_t(x, qw_ref[...]) + qb_ref[...]
    ko_ref[...] = _dot_t(x, kw_ref[...]) + kb_ref[...]
    wo_ref[...] = _dot_t(x, ww_ref[...]) + wb_ref[...]


def _qkv_body(x_ref, w_ref, b_ref, o_ref):
    o_ref[...] = _dot_t(x_ref[...], w_ref[...]) + b_ref[...]


def _attn_body(qi_ref, wi_ref, ki_ref, q_ref, k_ref, v_ref, o_ref, *, seq, bq, hd):
    wi = wi_ref[...]
    ki = ki_ref[...]
    scores = None
    for h in range(IH):
        dp = _dot_t(qi_ref[:, DX * h:DX * (h + 1)], ki)
        term = jnp.maximum(dp, 0.0) * wi[:, h:h + 1]
        scores = term if scores is None else scores + term

    bits = jax.lax.bitcast_convert_type(scores, jnp.int32)
    key_i = jnp.where(bits < 0, bits ^ jnp.int32(0x7FFFFFFF), bits)
    ukey = jax.lax.bitcast_convert_type(key_i, jnp.uint32) ^ jnp.uint32(0x80000000)
    t = jnp.zeros((bq, 1), jnp.uint32)
    for bit in range(31, -1, -1):
        cand = t | jnp.uint32(1 << bit)
        cnt = jnp.sum((ukey >= cand).astype(jnp.int32), axis=1, keepdims=True)
        t = jnp.where(cnt >= TOPK_K, cand, t)
    c_gt = jnp.sum((ukey > t).astype(jnp.int32), axis=1, keepdims=True)
    need = TOPK_K - c_gt
    tie = ukey == t
    iota = jax.lax.broadcasted_iota(jnp.int32, (bq, seq), 1)
    m = jnp.zeros((bq, 1), jnp.int32)
    for bit in range(12, -1, -1):
        cand = m | (1 << bit)
        cnt = jnp.sum((tie & (iota < cand)).astype(jnp.int32), axis=1,
                      keepdims=True)
        m = jnp.where(cnt <= need, cand, m)
    selected = (ukey > t) | (tie & (iota < m))
    bias = jnp.where(selected, 0.0, NEG_INF)

    scale = 1.0 / math.sqrt(hd)
    for h in range(NH):
        sl = slice(hd * h, hd * (h + 1))
        logits = _dot_t(q_ref[:, sl] * scale, k_ref[:, sl]) + bias
        mx = jnp.max(logits, axis=1, keepdims=True)
        p = jnp.exp(logits - mx)
        denom = jnp.sum(p, axis=1, keepdims=True)
        o_ref[:, sl] = _dot(p, v_ref[:, sl]) / denom


def _tail_body(att_ref, src_ref, ow_ref, ob_ref, l1w_ref, l1b_ref,
               l2w_ref, l2b_ref, g1_ref, b1_ref, g2_ref, b2_ref, o_ref):
    def ln(x, g, b):
        mu = jnp.mean(x, axis=1, keepdims=True)
        xc = x - mu
        var = jnp.mean(xc * xc, axis=1, keepdims=True)
        return xc * jax.lax.rsqrt(var + EPS) * g + b

    o = _dot_t(att_ref[...], ow_ref[...]) + ob_ref[...] + src_ref[...]
    x1 = ln(o, g1_ref[...], b1_ref[...])
    g = _dot_t(x1, l1w_ref[...]) + l1b_ref[...]
    gel = 0.5 * g * (1.0 + jax.lax.erf(g * (1.0 / math.sqrt(2.0))))
    y = _dot_t(gel, l2w_ref[...]) + l2b_ref[...] + x1
    o_ref[...] = ln(y, g2_ref[...], b2_ref[...])


def kernel(src, idx_qw, idx_qb, idx_kw, idx_kb, idx_ww, idx_wb, in_w, in_b,
           out_w, out_b, lin1_w, lin1_b, lin2_w, lin2_b, ln1_g, ln1_b,
           ln2_g, ln2_b):
    seq, batch, e = src.shape
    hd = e // NH
    bq = min(256, seq)
    nbq = seq // bq
    x = src.reshape(seq, e)
    f32 = jnp.float32

    r2 = lambda a: a.reshape(1, -1)

    qi, ki, wi = pl.pallas_call(
        _idx_proj_body,
        out_shape=[
            jax.ShapeDtypeStruct((seq, IH * DX), f32),
            jax.ShapeDtypeStruct((seq, DX), f32),
            jax.ShapeDtypeStruct((seq, IH), f32),
        ],
    )(x, idx_qw, r2(idx_qb), idx_kw, r2(idx_kb), idx_ww, r2(idx_wb))

    qkv = pl.pallas_call(
        _qkv_body,
        grid=(nbq,),
        in_specs=[
            pl.BlockSpec((bq, e), lambda i: (i, 0)),
            pl.BlockSpec((3 * e, e), lambda i: (0, 0)),
            pl.BlockSpec((1, 3 * e), lambda i: (0, 0)),
        ],
        out_specs=pl.BlockSpec((bq, 3 * e), lambda i: (i, 0)),
        out_shape=jax.ShapeDtypeStruct((seq, 3 * e), f32),
    )(x, in_w, r2(in_b))

    import functools
    attn = pl.pallas_call(
        functools.partial(_attn_body, seq=seq, bq=bq, hd=hd),
        grid=(nbq,),
        in_specs=[
            pl.BlockSpec((bq, IH * DX), lambda i: (i, 0)),
            pl.BlockSpec((bq, IH), lambda i: (i, 0)),
            pl.BlockSpec((seq, DX), lambda i: (0, 0)),
            pl.BlockSpec((bq, e), lambda i: (i, 0)),
            pl.BlockSpec((seq, e), lambda i: (0, 1)),
            pl.BlockSpec((seq, e), lambda i: (0, 2)),
        ],
        out_specs=pl.BlockSpec((bq, e), lambda i: (i, 0)),
        out_shape=jax.ShapeDtypeStruct((seq, e), f32),
    )(qi, wi, ki, qkv, qkv, qkv)

    dff = lin1_w.shape[0]
    out = pl.pallas_call(
        _tail_body,
        grid=(nbq,),
        in_specs=[
            pl.BlockSpec((bq, e), lambda i: (i, 0)),
            pl.BlockSpec((bq, e), lambda i: (i, 0)),
            pl.BlockSpec((e, e), lambda i: (0, 0)),
            pl.BlockSpec((1, e), lambda i: (0, 0)),
            pl.BlockSpec((dff, e), lambda i: (0, 0)),
            pl.BlockSpec((1, dff), lambda i: (0, 0)),
            pl.BlockSpec((e, dff), lambda i: (0, 0)),
            pl.BlockSpec((1, e), lambda i: (0, 0)),
            pl.BlockSpec((1, e), lambda i: (0, 0)),
            pl.BlockSpec((1, e), lambda i: (0, 0)),
            pl.BlockSpec((1, e), lambda i: (0, 0)),
            pl.BlockSpec((1, e), lambda i: (0, 0)),
        ],
        out_specs=pl.BlockSpec((bq, e), lambda i: (i, 0)),
        out_shape=jax.ShapeDtypeStruct((seq, e), f32),
    )(attn, x, out_w, r2(out_b), lin1_w, r2(lin1_b), lin2_w, r2(lin2_b),
      r2(ln1_g), r2(ln1_b), r2(ln2_g), r2(ln2_b))

    return out.reshape(seq, batch, e)
